# baseline (speedup 1.0000x reference)
import functools

import jax
import jax.numpy as jnp
from jax import lax
from jax.experimental import pallas as pl
from jax.experimental.pallas import tpu as pltpu
from jax.experimental.pallas import tpu_sc as plsc

PLACEHOLDER = -1
TINY = float(jnp.finfo(jnp.float32).tiny)
LANES = 16


def _sc_gather_body(V, tok_hbm, tl_hbm, dp_hbm, tltok_hbm, dptok_hbm,
                    tok_v, idx_v, val_v, sem):
    c = lax.axis_index("c")
    s = lax.axis_index("s")
    wid = s * 2 + c
    g = wid % 16
    base = g * LANES
    pltpu.sync_copy(tok_hbm.at[pl.ds(base, LANES)], tok_v)
    rows = base + lax.broadcasted_iota(jnp.int32, (LANES,), 0)
    idx_v[...] = rows * V + tok_v[...]

    @pl.when(wid < 16)
    def _():
        cp = pltpu.make_async_copy(tl_hbm.at[idx_v], val_v, sem)
        cp.start()
        cp.wait()
        pltpu.sync_copy(val_v, tltok_hbm.at[pl.ds(base, LANES)])

    @pl.when(wid >= 16)
    def _():
        cp = pltpu.make_async_copy(dp_hbm.at[idx_v], val_v, sem)
        cp.start()
        cp.wait()
        pltpu.sync_copy(val_v, dptok_hbm.at[pl.ds(base, LANES)])


def _sc_gather(tok_flat, tl_flat, dp_flat, V):
    mesh = plsc.VectorSubcoreMesh(core_axis_name="c", subcore_axis_name="s",
                                  num_cores=2, num_subcores=16)
    f = pl.kernel(
        functools.partial(_sc_gather_body, V),
        out_type=(jax.ShapeDtypeStruct(tok_flat.shape, jnp.float32),
                  jax.ShapeDtypeStruct(tok_flat.shape, jnp.float32)),
        mesh=mesh,
        scratch_types=[
            pltpu.VMEM((LANES,), jnp.int32),
            pltpu.VMEM((LANES,), jnp.int32),
            pltpu.VMEM((LANES,), jnp.float32),
            pltpu.SemaphoreType.DMA,
        ],
    )
    return f(tok_flat, tl_flat, dp_flat)


def _tc_body(temp_s, tok_s, bonus_s, tl_ref, dp_hbm, q_ref, tltok_ref,
             dptok_ref, u_ref, out_ref, dp_row, sem_d):
    _, LS, VS = tl_ref.shape
    SUB = 8
    L = LS // SUB
    V = SUB * VS
    b = pl.program_id(0)
    temp = temp_s[b]
    xg = tl_ref[0].reshape(L, SUB, VS)
    scaled = xg / temp
    m2 = jnp.max(scaled, axis=2)
    m = jnp.max(m2, axis=1, keepdims=True)
    e = jnp.exp(scaled - m.reshape(L, 1, 1))
    s2 = jnp.sum(e, axis=2)
    s = jnp.sum(s2, axis=1, keepdims=True)

    tltok_col = tltok_ref[b]
    dptok_col = dptok_ref[b]
    tp_tok = jnp.exp(tltok_col / temp - m) / s
    u_col = u_ref[b]
    accept = (tp_tok / dptok_col) >= u_col
    iota8 = jax.lax.broadcasted_iota(jnp.int32, (L, 1), 0)
    n = jnp.min(jnp.where(accept, L, iota8))
    r = jnp.minimum(n, L - 1)

    cpd = pltpu.make_async_copy(dp_hbm.at[b, pl.ds(r * SUB, SUB), :],
                                dp_row, sem_d)
    cpd.start()
    m_r = jnp.sum(jnp.where(iota8 == r, m, 0.0))
    s_r = jnp.sum(jnp.where(iota8 == r, s, 0.0))
    tl_r = tl_ref[0, pl.ds(r * SUB, SUB), :]
    e_r = jnp.exp(tl_r / temp - m_r)
    cpd.wait()
    dp_r = dp_row[...]
    padj = jnp.maximum(e_r / s_r - dp_r, TINY)
    S = jnp.sum(padj)
    q = q_ref[0]
    ratio = (padj / S) / q
    mx = jnp.max(ratio)
    iota_flat = (jax.lax.broadcasted_iota(jnp.int32, (SUB, VS), 0) * VS
                 + jax.lax.broadcasted_iota(jnp.int32, (SUB, VS), 1))
    rec_at = jnp.min(jnp.where(ratio == mx, iota_flat, V))

    fill = jnp.where(n < L, rec_at, bonus_s[b]).astype(jnp.int32)
    for j in range(L + 1):
        tok_j = tok_s[b, j] if j < L else jnp.int32(0)
        v = jnp.where(j < n, tok_j,
                      jnp.where(j == n, fill, jnp.int32(PLACEHOLDER)))
        out_ref[b, j] = v.astype(jnp.int32)


def kernel(draft_token_ids, cu_num_draft_tokens, draft_probs, target_logits,
           bonus_token_ids, temperature, uniform_probs, q_exp):
    B, L = draft_token_ids.shape
    V = target_logits.shape[-1]
    tok_flat = draft_token_ids.reshape(B * L)
    tl_flat = target_logits.reshape(B * L * V)
    dp_flat = draft_probs.reshape(B * L * V)
    tltok, dptok = _sc_gather(tok_flat, tl_flat, dp_flat, V)

    SUB = 8
    VS = V // SUB
    q4 = q_exp.reshape(B, SUB, VS)
    tl5 = target_logits.reshape(B, L * SUB, VS)
    dp5 = draft_probs.reshape(B, L * SUB, VS)
    u3 = uniform_probs.reshape(B, L, 1)
    tltok3 = tltok.reshape(B, L, 1)
    dptok3 = dptok.reshape(B, L, 1)

    out = pl.pallas_call(
        _tc_body,
        grid=(B,),
        in_specs=[
            pl.BlockSpec(memory_space=pltpu.SMEM),
            pl.BlockSpec(memory_space=pltpu.SMEM),
            pl.BlockSpec(memory_space=pltpu.SMEM),
            pl.BlockSpec((1, L * SUB, VS), lambda b: (b, 0, 0)),
            pl.BlockSpec(memory_space=pltpu.MemorySpace.HBM),
            pl.BlockSpec((1, SUB, VS), lambda b: (b, 0, 0)),
            pl.BlockSpec((B, L, 1), lambda b: (0, 0, 0)),
            pl.BlockSpec((B, L, 1), lambda b: (0, 0, 0)),
            pl.BlockSpec((B, L, 1), lambda b: (0, 0, 0)),
        ],
        out_specs=pl.BlockSpec(memory_space=pltpu.SMEM),
        out_shape=jax.ShapeDtypeStruct((B, L + 1), jnp.int32),
        scratch_shapes=[
            pltpu.VMEM((SUB, VS), jnp.float32),
            pltpu.SemaphoreType.DMA,
        ],
        compiler_params=pltpu.CompilerParams(
            dimension_semantics=("arbitrary",)),
    )(temperature, draft_token_ids, bonus_token_ids,
      tl5, dp5, q4, tltok3, dptok3, u3)
    return out


# speedup vs baseline: 2.3733x; 2.3733x over previous
import jax
import jax.numpy as jnp
from jax.experimental import pallas as pl
from jax.experimental.pallas import tpu as pltpu

PLACEHOLDER = -1
TINY = float(jnp.finfo(jnp.float32).tiny)


def _body(temp_s, tok_s, bonus_s, tl_ref, dp_ref, q_ref, tok_ref, u_ref,
          out_ref):
    L, V = tl_ref.shape
    b = pl.program_id(0)
    temp = temp_s[b]
    tl = tl_ref[...]
    scaled = tl / temp
    m = jnp.max(scaled, axis=1, keepdims=True)
    e = jnp.exp(scaled - m)
    s = jnp.sum(e, axis=1, keepdims=True)

    dp = dp_ref[0]
    iota_v = jax.lax.broadcasted_iota(jnp.int32, (L, V), 1)
    tok_col = tok_ref[b]
    mask = iota_v == tok_col
    e_tok = jnp.sum(jnp.where(mask, e, 0.0), axis=1, keepdims=True)
    dp_tok = jnp.sum(jnp.where(mask, dp, 0.0), axis=1, keepdims=True)
    tp_tok = e_tok / s
    u_col = u_ref[b]
    accept = (tp_tok / dp_tok) >= u_col
    iota8 = jax.lax.broadcasted_iota(jnp.int32, (L, 1), 0)
    n = jnp.min(jnp.where(accept, L, iota8))
    r = jnp.minimum(n, L - 1)

    m_r = jnp.sum(jnp.where(iota8 == r, m, 0.0))
    s_r = jnp.sum(jnp.where(iota8 == r, s, 0.0))
    tl_r = tl_ref[pl.ds(r, 1), :]
    e_r = jnp.exp(tl_r / temp - m_r)
    dp_r = dp_ref[0, pl.ds(r, 1), :]
    padj = jnp.maximum(e_r / s_r - dp_r, TINY)
    S = jnp.sum(padj)
    q_row = q_ref[pl.ds(b, 1), :]
    ratio = (padj / S) / q_row
    mx = jnp.max(ratio)
    iota_v1 = jax.lax.broadcasted_iota(jnp.int32, (1, V), 1)
    rec_at = jnp.min(jnp.where(ratio == mx, iota_v1, V))

    fill = jnp.where(n < L, rec_at, bonus_s[b]).astype(jnp.int32)
    for j in range(L + 1):
        tok_j = tok_s[b, j] if j < L else jnp.int32(0)
        v = jnp.where(j < n, tok_j,
                      jnp.where(j == n, fill, jnp.int32(PLACEHOLDER)))
        out_ref[b, j] = v.astype(jnp.int32)


def kernel(draft_token_ids, cu_num_draft_tokens, draft_probs, target_logits,
           bonus_token_ids, temperature, uniform_probs, q_exp):
    B, L = draft_token_ids.shape
    V = target_logits.shape[-1]
    tok3 = draft_token_ids.reshape(B, L, 1)
    u3 = uniform_probs.reshape(B, L, 1)

    out = pl.pallas_call(
        _body,
        grid=(B,),
        in_specs=[
            pl.BlockSpec(memory_space=pltpu.SMEM),
            pl.BlockSpec(memory_space=pltpu.SMEM),
            pl.BlockSpec(memory_space=pltpu.SMEM),
            pl.BlockSpec((L, V), lambda b: (b, 0)),
            pl.BlockSpec((1, L, V), lambda b: (b, 0, 0)),
            pl.BlockSpec((B, V), lambda b: (0, 0)),
            pl.BlockSpec((B, L, 1), lambda b: (0, 0, 0)),
            pl.BlockSpec((B, L, 1), lambda b: (0, 0, 0)),
        ],
        out_specs=pl.BlockSpec(memory_space=pltpu.SMEM),
        out_shape=jax.ShapeDtypeStruct((B, L + 1), jnp.int32),
        compiler_params=pltpu.CompilerParams(
            dimension_semantics=("arbitrary",)),
    )(temperature, draft_token_ids, bonus_token_ids,
      target_logits, draft_probs, q_exp, tok3, u3)
    return out
